# baseline (speedup 1.0000x reference)
import functools
import math

import numpy as np
import jax
import jax.numpy as jnp
from jax import lax
from jax.experimental import pallas as pl
from jax.experimental.pallas import tpu as pltpu
from jax.experimental.pallas import tpu_sc as plsc


def _positional_encoding(length, depth):
    depth = depth / 2
    positions = np.arange(length)[:, np.newaxis]
    depths = np.arange(depth)[np.newaxis, :] / depth
    angle_rates = 1 / 10000 ** depths
    angle_rads = positions * angle_rates
    return np.concatenate([np.sin(angle_rads), np.cos(angle_rads)], axis=-1).astype(np.float32)


def _sc_geometry():
    try:
        info = plsc.get_sparse_core_info()
        return info.num_cores, info.num_subcores
    except Exception:
        return 2, 16


@functools.lru_cache(maxsize=None)
def _build(B, S, V, D):
    NC, NS = _sc_geometry()
    NW = NC * NS
    ROWS = B * S
    assert ROWS % NW == 0
    per_w = ROWS // NW
    CH = 2 * S
    assert per_w % CH == 0
    NCH = per_w // CH
    assert D % 16 == 0
    DV = D // 16
    scale = float(math.sqrt(float(D)))

    slices = []
    off = 0
    while off < CH:
        ln = min(128, CH - off)
        slices.append((off, ln))
        off += ln

    mesh = plsc.VectorSubcoreMesh(core_axis_name="c", subcore_axis_name="s")

    @functools.partial(
        pl.kernel,
        mesh=mesh,
        compiler_params=pltpu.CompilerParams(use_tc_tiling_on_sc=False),
        out_type=jax.ShapeDtypeStruct((ROWS, D), jnp.float32),
        scratch_types=[
            pltpu.VMEM((S, D), jnp.float32),
            pltpu.VMEM((CH, D), jnp.float32),
            pltpu.VMEM((CH,), jnp.int32),
            pltpu.SemaphoreType.DMA,
        ],
    )
    def _k(table_hbm, xf_hbm, pos_hbm, out_hbm, pos_v, rows_v, idx_v, sem):
        wid = lax.axis_index("s") * NC + lax.axis_index("c")
        base = wid * per_w
        pltpu.sync_copy(pos_hbm, pos_v)

        def chunk_body(c, carry):
            row0 = base + c * CH
            pltpu.sync_copy(xf_hbm.at[pl.ds(row0, CH)], idx_v)
            copies = [
                pltpu.async_copy(
                    table_hbm.at[idx_v.at[pl.ds(o, ln)]],
                    rows_v.at[pl.ds(o, ln)],
                    sem,
                )
                for (o, ln) in slices
            ]
            for cp in copies:
                cp.wait()

            def srow(s, carry2):
                for cc in range(DV):
                    p = pos_v[s, pl.ds(cc * 16, 16)]
                    for q in range(CH // S):
                        r = q * S + s
                        rows_v[r, pl.ds(cc * 16, 16)] = (
                            rows_v[r, pl.ds(cc * 16, 16)] * scale + p
                        )
                return carry2

            lax.fori_loop(0, S, srow, 0)
            pltpu.sync_copy(rows_v, out_hbm.at[pl.ds(row0, CH)])
            return carry

        lax.fori_loop(0, NCH, chunk_body, 0)

    return _k


def kernel(x, embedding_table):
    B, S = x.shape
    V, D = embedding_table.shape
    pos = jnp.asarray(_positional_encoding(S, D))
    x_flat = x.reshape(-1).astype(jnp.int32)
    out = _build(B, S, V, D)(embedding_table, x_flat, pos)
    return out.reshape(B, S, D)
